# initial kernel scaffold (unmeasured)
import jax
import jax.numpy as jnp
from jax import lax
from jax.experimental import pallas as pl
from jax.experimental.pallas import tpu as pltpu

N_DEV = 4
SQ = 256
SKV_LOCAL = 4096
HQ = 8
HKV = 2
DH = 128
DMODEL = 1024
SCALE = 0.08838834764831843
KV_CHUNK = 1024

ROWS_O = HQ * SQ
ROWS_ML = ROWS_O // 128
ROWS_TOT = ROWS_O + 2 * ROWS_ML


def _combine(o_a, m_a, l_a, o_b, m_b, l_b):
    m_n = jnp.maximum(m_a, m_b)
    a = jnp.exp(m_a - m_n)
    b = jnp.exp(m_b - m_n)
    return o_a * a + o_b * b, m_n, l_a * a + l_b * b


def kernel(x, Wq, Wo, K_ext, V_ext):
    x2 = x.reshape(SQ, DMODEL)
    K = jnp.transpose(K_ext[0], (1, 0, 2))
    V = jnp.transpose(V_ext[0], (1, 0, 2))

    def body(x_ref, wq_ref, wo_ref, k_ref, v_ref, out_ref,
             comm_ref, send_sems, recv_sems):
        my = lax.axis_index("i")
        left = lax.rem(my + N_DEV - 1, N_DEV)
        right = lax.rem(my + 1, N_DEV)

        barrier_sem = pltpu.get_barrier_semaphore()
        for nbr in (left, right):
            pl.semaphore_signal(
                barrier_sem, inc=1,
                device_id=(nbr,), device_id_type=pl.DeviceIdType.MESH,
            )
        pl.semaphore_wait(barrier_sem, 2)

        xv = x_ref[:, :]
        o_parts, m_parts, l_parts = [], [], []
        for kvh in range(HKV):
            qg = jnp.concatenate(
                [
                    jnp.dot(
                        xv,
                        wq_ref[:, (4 * kvh + g) * DH:(4 * kvh + g + 1) * DH],
                        preferred_element_type=jnp.float32,
                    )
                    for g in range(4)
                ],
                axis=0,
            ) * SCALE

            m = l = o = None
            for c in range(SKV_LOCAL // KV_CHUNK):
                kc = k_ref[kvh, c * KV_CHUNK:(c + 1) * KV_CHUNK, :]
                vc = v_ref[kvh, c * KV_CHUNK:(c + 1) * KV_CHUNK, :]
                s = lax.dot_general(
                    qg, kc,
                    dimension_numbers=(((1,), (1,)), ((), ())),
                    preferred_element_type=jnp.float32,
                )
                mj = jnp.max(s, axis=1, keepdims=True)
                if c == 0:
                    m_n = mj
                    p = jnp.exp(s - m_n)
                    l = jnp.sum(p, axis=1, keepdims=True)
                    o = jnp.dot(p, vc, preferred_element_type=jnp.float32)
                else:
                    m_n = jnp.maximum(m, mj)
                    alpha = jnp.exp(m - m_n)
                    p = jnp.exp(s - m_n)
                    l = l * alpha + jnp.sum(p, axis=1, keepdims=True)
                    o = o * alpha + jnp.dot(
                        p, vc, preferred_element_type=jnp.float32)
                m = m_n
            o_parts.append(o)
            m_parts.append(m)
            l_parts.append(l)

        o_acc = jnp.concatenate(o_parts, axis=0)
        m_acc = jnp.concatenate(m_parts, axis=0)
        l_acc = jnp.concatenate(l_parts, axis=0)

        comm_ref[0, 0:ROWS_O, :] = o_acc
        comm_ref[0, ROWS_O:ROWS_O + ROWS_ML, :] = m_acc.reshape(ROWS_ML, 128)
        comm_ref[0, ROWS_O + ROWS_ML:ROWS_TOT, :] = l_acc.reshape(ROWS_ML, 128)

        for h in range(N_DEV - 1):
            rdma = pltpu.make_async_remote_copy(
                src_ref=comm_ref.at[h],
                dst_ref=comm_ref.at[h + 1],
                send_sem=send_sems.at[h],
                recv_sem=recv_sems.at[h],
                device_id=(right,),
                device_id_type=pl.DeviceIdType.MESH,
            )
            rdma.start()
            rdma.wait()

            o_r = comm_ref[h + 1, 0:ROWS_O, :]
            m_r = comm_ref[h + 1, ROWS_O:ROWS_O + ROWS_ML, :].reshape(
                ROWS_O, 1)
            l_r = comm_ref[h + 1, ROWS_O + ROWS_ML:ROWS_TOT, :].reshape(
                ROWS_O, 1)
            o_acc, m_acc, l_acc = _combine(o_acc, m_acc, l_acc, o_r, m_r, l_r)

        o_n = o_acc / l_acc
        attn = jnp.concatenate(
            [o_n[hh * SQ:(hh + 1) * SQ, :] for hh in range(HQ)], axis=1
        )
        out_ref[:, :] = jnp.dot(
            attn, wo_ref[:, :], preferred_element_type=jnp.float32)

    out2 = pl.pallas_call(
        body,
        out_shape=jax.ShapeDtypeStruct((SQ, DMODEL), jnp.float32),
        in_specs=[pl.BlockSpec(memory_space=pltpu.VMEM)] * 5,
        out_specs=pl.BlockSpec(memory_space=pltpu.VMEM),
        scratch_shapes=[
            pltpu.VMEM((N_DEV, ROWS_TOT, 128), jnp.float32),
            pltpu.SemaphoreType.DMA((N_DEV - 1,)),
            pltpu.SemaphoreType.DMA((N_DEV - 1,)),
        ],
        compiler_params=pltpu.CompilerParams(collective_id=0),
    )(x2, Wq, Wo, K, V)

    return out2.reshape(1, SQ, DMODEL)


# baseline (device time: 141396 ns/iter reference)
import jax
import jax.numpy as jnp
from jax import lax
from jax.experimental import pallas as pl
from jax.experimental.pallas import tpu as pltpu

N_DEV = 4
SQ = 256
SKV_LOCAL = 4096
HQ = 8
HKV = 2
DH = 128
DMODEL = 1024
SCALE = 0.08838834764831843
KV_CHUNK = 1024

ROWS_O = HQ * SQ


def _combine(o_a, m_a, l_a, o_b, m_b, l_b):
    m_n = jnp.maximum(m_a, m_b)
    a = jnp.exp(m_a - m_n)
    b = jnp.exp(m_b - m_n)
    return o_a * a + o_b * b, m_n, l_a * a + l_b * b


def kernel(x, Wq, Wo, K_ext, V_ext):
    x2 = x.reshape(SQ, DMODEL)
    K = jnp.transpose(K_ext[0], (1, 0, 2))
    V = jnp.transpose(V_ext[0], (1, 0, 2))

    def body(x_ref, wq_ref, wo_ref, k_ref, v_ref, out_ref,
             comm_ref, ml_ref, send_sems, recv_sems,
             ml_send_sems, ml_recv_sems):
        my = lax.axis_index("i")
        left = lax.rem(my + N_DEV - 1, N_DEV)
        right = lax.rem(my + 1, N_DEV)

        barrier_sem = pltpu.get_barrier_semaphore()
        for nbr in (left, right):
            pl.semaphore_signal(
                barrier_sem, inc=1,
                device_id=(nbr,), device_id_type=pl.DeviceIdType.MESH,
            )
        pl.semaphore_wait(barrier_sem, 2)

        xv = x_ref[:, :]
        o_parts, m_parts, l_parts = [], [], []
        for kvh in range(HKV):
            qg = jnp.concatenate(
                [
                    jnp.dot(
                        xv,
                        wq_ref[:, (4 * kvh + g) * DH:(4 * kvh + g + 1) * DH],
                        preferred_element_type=jnp.float32,
                    )
                    for g in range(4)
                ],
                axis=0,
            ) * SCALE

            m = l = o = None
            for c in range(SKV_LOCAL // KV_CHUNK):
                kc = k_ref[kvh, c * KV_CHUNK:(c + 1) * KV_CHUNK, :]
                vc = v_ref[kvh, c * KV_CHUNK:(c + 1) * KV_CHUNK, :]
                s = lax.dot_general(
                    qg, kc,
                    dimension_numbers=(((1,), (1,)), ((), ())),
                    preferred_element_type=jnp.float32,
                )
                mj = jnp.max(s, axis=1, keepdims=True)
                if c == 0:
                    m_n = mj
                    p = jnp.exp(s - m_n)
                    l = jnp.sum(p, axis=1, keepdims=True)
                    o = jnp.dot(p, vc, preferred_element_type=jnp.float32)
                else:
                    m_n = jnp.maximum(m, mj)
                    alpha = jnp.exp(m - m_n)
                    p = jnp.exp(s - m_n)
                    l = l * alpha + jnp.sum(p, axis=1, keepdims=True)
                    o = o * alpha + jnp.dot(
                        p, vc, preferred_element_type=jnp.float32)
                m = m_n
            o_parts.append(o)
            m_parts.append(m)
            l_parts.append(l)

        o_acc = jnp.concatenate(o_parts, axis=0)
        m_acc = jnp.concatenate(m_parts, axis=0)
        l_acc = jnp.concatenate(l_parts, axis=0)

        comm_ref[0, :, :] = o_acc
        ml_ref[0, 0, :, :] = m_acc
        ml_ref[0, 1, :, :] = l_acc

        for h in range(N_DEV - 1):
            rdma_o = pltpu.make_async_remote_copy(
                src_ref=comm_ref.at[h],
                dst_ref=comm_ref.at[h + 1],
                send_sem=send_sems.at[h],
                recv_sem=recv_sems.at[h],
                device_id=(right,),
                device_id_type=pl.DeviceIdType.MESH,
            )
            rdma_ml = pltpu.make_async_remote_copy(
                src_ref=ml_ref.at[h],
                dst_ref=ml_ref.at[h + 1],
                send_sem=ml_send_sems.at[h],
                recv_sem=ml_recv_sems.at[h],
                device_id=(right,),
                device_id_type=pl.DeviceIdType.MESH,
            )
            rdma_o.start()
            rdma_ml.start()
            rdma_o.wait()
            rdma_ml.wait()

            o_r = comm_ref[h + 1, :, :]
            m_r = ml_ref[h + 1, 0, :, :]
            l_r = ml_ref[h + 1, 1, :, :]
            o_acc, m_acc, l_acc = _combine(o_acc, m_acc, l_acc, o_r, m_r, l_r)

        o_n = o_acc / l_acc
        attn = jnp.concatenate(
            [o_n[hh * SQ:(hh + 1) * SQ, :] for hh in range(HQ)], axis=1
        )
        out_ref[:, :] = jnp.dot(
            attn, wo_ref[:, :], preferred_element_type=jnp.float32)

    out2 = pl.pallas_call(
        body,
        out_shape=jax.ShapeDtypeStruct((SQ, DMODEL), jnp.float32),
        in_specs=[pl.BlockSpec(memory_space=pltpu.VMEM)] * 5,
        out_specs=pl.BlockSpec(memory_space=pltpu.VMEM),
        scratch_shapes=[
            pltpu.VMEM((N_DEV, ROWS_O, 128), jnp.float32),
            pltpu.VMEM((N_DEV, 2, ROWS_O, 1), jnp.float32),
            pltpu.SemaphoreType.DMA((N_DEV - 1,)),
            pltpu.SemaphoreType.DMA((N_DEV - 1,)),
            pltpu.SemaphoreType.DMA((N_DEV - 1,)),
            pltpu.SemaphoreType.DMA((N_DEV - 1,)),
        ],
        compiler_params=pltpu.CompilerParams(
            collective_id=0,
            vmem_limit_bytes=100 * 1024 * 1024,
        ),
    )(x2, Wq, Wo, K, V)

    return out2.reshape(1, SQ, DMODEL)
